# baseline (speedup 1.0000x reference)
import functools

import jax
import jax.numpy as jnp
from jax import lax
from jax.experimental import pallas as pl
from jax.experimental.pallas import tpu as pltpu
from jax.experimental.pallas import tpu_sc as plsc

N_SPOTS = 50000
D = 128
N_GENES = 512
K = 64

_LOGITS_BLOCK = 10240
_N_PAD = 5 * _LOGITS_BLOCK


def _logits_body(x_ref, w_ref, b_ref, v_ref, o_ref):
    x = x_ref[...]
    h = jnp.tanh(
        lax.dot_general(
            x, w_ref[...], (((1,), (1,)), ((), ())),
            preferred_element_type=jnp.float32,
        )
        + b_ref[...]
    )
    o_ref[...] = jnp.sum(h * v_ref[...], axis=1)


def _spot_logits(spot_emb, W_w, W_b2, v_w):
    return pl.pallas_call(
        _logits_body,
        grid=(5,),
        in_specs=[
            pl.BlockSpec((_LOGITS_BLOCK, D), lambda i: (i, 0)),
            pl.BlockSpec((D, D), lambda i: (0, 0)),
            pl.BlockSpec((1, D), lambda i: (0, 0)),
            pl.BlockSpec((1, D), lambda i: (0, 0)),
        ],
        out_specs=pl.BlockSpec((_LOGITS_BLOCK,), lambda i: (i,)),
        out_shape=jax.ShapeDtypeStruct((_N_PAD,), jnp.float32),
    )(spot_emb, W_w, W_b2, v_w)


_NTILES = 32
_GPT = N_GENES // _NTILES


def _sc_pool_body(mask_hbm, logits_hbm, emb_hbm, out_hbm,
                  ids_v, lg_v, wts_v, out_v, bufa_v, bufb_v,
                  sem_l, sem_a, sem_b):
    nc = 2
    wid = lax.axis_index("s") * nc + lax.axis_index("c")

    pltpu.sync_copy(mask_hbm.at[pl.ds(wid * _GPT, _GPT)], ids_v)
    lcps = [pltpu.async_copy(logits_hbm.at[ids_v.at[g]], lg_v.at[g], sem_l)
            for g in range(_GPT)]
    pltpu.async_copy(emb_hbm.at[ids_v.at[0]], bufa_v, sem_a)
    for cp in lcps:
        cp.wait()

    def phase1(table_v):
        def gene_w(g, carry):
            ids = [ids_v[g, pl.ds(16 * q, 16)] for q in range(4)]
            pay = [lax.iota(jnp.int32, 16) + 16 * q for q in range(4)]
            for q in range(4):
                plsc.store_scatter(table_v, [ids[q]], pay[q])
            valid = [plsc.load_gather(table_v, [ids[q]]) == pay[q]
                     for q in range(4)]
            ls = [lg_v[g, pl.ds(16 * q, 16)] for q in range(4)]
            lm = [jnp.where(valid[q], ls[q], jnp.float32(-1e30))
                  for q in range(4)]
            m = jnp.max(jnp.maximum(jnp.maximum(lm[0], lm[1]),
                                    jnp.maximum(lm[2], lm[3])))
            es = [jnp.where(valid[q], jnp.exp(ls[q] - m), jnp.float32(0.0))
                  for q in range(4)]
            s = jnp.sum(es[0] + es[1] + es[2] + es[3])
            inv_v = jnp.full((16,), 1.0, jnp.float32) / jnp.full(
                (16,), s, jnp.float32)
            for q in range(4):
                wts_v[g, pl.ds(16 * q, 16)] = es[q] * inv_v
            return carry

        lax.fori_loop(0, _GPT, gene_w, jnp.int32(0))

    pl.run_scoped(phase1, pltpu.VMEM((N_SPOTS,), jnp.int32))

    def pool_gene(g, buf):
        ws = [wts_v[g, pl.ds(16 * q, 16)] for q in range(4)]
        accs = [jnp.zeros((16,), jnp.float32) for _ in range(8)]
        for q in range(4):
            for u in range(16):
                ek = ws[q][u]
                for j in range(8):
                    accs[j] = accs[j] + ek * buf[q * 16 + u,
                                                 pl.ds(16 * j, 16)]
        for j in range(8):
            out_v[g, pl.ds(16 * j, 16)] = accs[j]

    def gene_step(g, carry):
        even = lax.rem(g, 2) == 0

        @pl.when(even)
        def _():
            pltpu.make_async_copy(emb_hbm.at[ids_v.at[0]], bufa_v,
                                  sem_a).wait()
            @pl.when(g + 1 < _GPT)
            def _():
                pltpu.async_copy(emb_hbm.at[ids_v.at[g + 1]], bufb_v, sem_b)
            pool_gene(g, bufa_v)

        @pl.when(jnp.logical_not(even))
        def _():
            pltpu.make_async_copy(emb_hbm.at[ids_v.at[0]], bufb_v,
                                  sem_b).wait()
            @pl.when(g + 1 < _GPT)
            def _():
                pltpu.async_copy(emb_hbm.at[ids_v.at[g + 1]], bufa_v, sem_a)
            pool_gene(g, bufb_v)

        return carry

    lax.fori_loop(0, _GPT, gene_step, jnp.int32(0))

    pltpu.sync_copy(out_v, out_hbm.at[pl.ds(wid * _GPT, _GPT)])


def _sc_pool(mask2d, logits, spot_emb):
    mesh = plsc.VectorSubcoreMesh(core_axis_name="c", subcore_axis_name="s")
    f = pl.kernel(
        _sc_pool_body,
        out_type=jax.ShapeDtypeStruct((N_GENES, D), jnp.float32),
        mesh=mesh,
        scratch_types=[
            pltpu.VMEM((_GPT, K), jnp.int32),
            pltpu.VMEM((_GPT, K), jnp.float32),
            pltpu.VMEM((_GPT, K), jnp.float32),
            pltpu.VMEM((_GPT, D), jnp.float32),
            pltpu.VMEM((K, D), jnp.float32),
            pltpu.VMEM((K, D), jnp.float32),
            pltpu.SemaphoreType.DMA,
            pltpu.SemaphoreType.DMA,
            pltpu.SemaphoreType.DMA,
        ],
        compiler_params=pltpu.CompilerParams(needs_layout_passes=False),
    )
    return f(mask2d, logits, spot_emb)


def kernel(spot_emb, gene_spot_mask, W_w, W_b, v_w):
    logits = _spot_logits(spot_emb, W_w, W_b.reshape(1, D),
                          v_w.reshape(1, D))
    return _sc_pool(gene_spot_mask.astype(jnp.int32), logits, spot_emb)


# speedup vs baseline: 1.1855x; 1.1855x over previous
import functools

import jax
import jax.numpy as jnp
from jax import lax
from jax.experimental import pallas as pl
from jax.experimental.pallas import tpu as pltpu
from jax.experimental.pallas import tpu_sc as plsc

N_SPOTS = 50000
D = 128
N_GENES = 512
K = 64

_LOGITS_BLOCK = 10240
_N_PAD = 5 * _LOGITS_BLOCK


def _logits_body(x_ref, w_ref, b_ref, v_ref, o_ref):
    x = x_ref[...]
    h = jnp.tanh(
        lax.dot_general(
            x, w_ref[...], (((1,), (1,)), ((), ())),
            preferred_element_type=jnp.float32,
        )
        + b_ref[...]
    )
    o_ref[...] = jnp.sum(h * v_ref[...], axis=1)


def _spot_logits(spot_emb, W_w, W_b2, v_w):
    return pl.pallas_call(
        _logits_body,
        grid=(5,),
        in_specs=[
            pl.BlockSpec((_LOGITS_BLOCK, D), lambda i: (i, 0)),
            pl.BlockSpec((D, D), lambda i: (0, 0)),
            pl.BlockSpec((1, D), lambda i: (0, 0)),
            pl.BlockSpec((1, D), lambda i: (0, 0)),
        ],
        out_specs=pl.BlockSpec((_LOGITS_BLOCK,), lambda i: (i,)),
        out_shape=jax.ShapeDtypeStruct((_N_PAD,), jnp.float32),
    )(spot_emb, W_w, W_b2, v_w)


_NTILES = 32
_GPT = N_GENES // _NTILES
_SLOTS = _GPT * K
_CHUNK_GENES = 4
_ROWS = _CHUNK_GENES * K
_NCHUNKS = _GPT // _CHUNK_GENES
_IDXCH = 128


def _sc_pool_body(mask_hbm, logits_hbm, emb_hbm, out_hbm,
                  ids2_v, ids_v, lg_v, val_v, wts_v, out_v,
                  sem_l, sem_r0, sem_r1):
    nc = 2
    wid = lax.axis_index("s") * nc + lax.axis_index("c")

    pltpu.sync_copy(mask_hbm.at[pl.ds(wid * _GPT, _GPT)], ids2_v)
    for r in range(_GPT):
        for q in range(4):
            ids_v[pl.ds(r * K + 16 * q, 16)] = ids2_v[r, pl.ds(16 * q, 16)]

    lcps = [
        pltpu.async_copy(
            logits_hbm.at[ids_v.at[pl.ds(i * _IDXCH, _IDXCH)]],
            lg_v.at[pl.ds(i * _IDXCH, _IDXCH)], sem_l)
        for i in range(_SLOTS // _IDXCH)
    ]

    def dedup(table_v):
        def gene_d(g, carry):
            sb = g * K
            ids = [ids_v[pl.ds(sb + 16 * q, 16)] for q in range(4)]
            pay = [lax.iota(jnp.int32, 16) + 16 * q for q in range(4)]
            for q in range(4):
                plsc.store_scatter(table_v, [ids[q]], pay[q])
            for q in range(4):
                ok = plsc.load_gather(table_v, [ids[q]]) == pay[q]
                val_v[pl.ds(sb + 16 * q, 16)] = jnp.where(
                    ok, jnp.float32(1.0), jnp.float32(0.0))
            return carry

        lax.fori_loop(0, _GPT, gene_d, jnp.int32(0))

    pl.run_scoped(dedup, pltpu.VMEM((N_SPOTS,), jnp.int32))

    def phase2(rows0_v, rows1_v):
        rows = [rows0_v, rows1_v]
        sems = [sem_r0, sem_r1]

        def issue_rows(c):
            buf = rows[c % 2]
            sem = sems[c % 2]
            return [
                pltpu.async_copy(
                    emb_hbm.at[ids_v.at[pl.ds(c * _ROWS + i * _IDXCH,
                                              _IDXCH)]],
                    buf.at[pl.ds(i * _IDXCH, _IDXCH)], sem)
                for i in range(_ROWS // _IDXCH)
            ]

        rcps = issue_rows(0)
        for cp in lcps:
            cp.wait()

        def gene_w(g, carry):
            sb = g * K
            ls = [lg_v[pl.ds(sb + 16 * q, 16)] for q in range(4)]
            va = [val_v[pl.ds(sb + 16 * q, 16)] > jnp.float32(0.5)
                  for q in range(4)]
            lm = [jnp.where(va[q], ls[q], jnp.float32(-1e30))
                  for q in range(4)]
            m = jnp.max(jnp.maximum(jnp.maximum(lm[0], lm[1]),
                                    jnp.maximum(lm[2], lm[3])))
            es = [jnp.where(va[q], jnp.exp(ls[q] - m), jnp.float32(0.0))
                  for q in range(4)]
            s = jnp.sum(es[0] + es[1] + es[2] + es[3])
            inv_v = jnp.full((16,), 1.0, jnp.float32) / jnp.full(
                (16,), s, jnp.float32)
            for q in range(4):
                wts_v[pl.ds(sb + 16 * q, 16)] = es[q] * inv_v
            return carry

        lax.fori_loop(0, _GPT, gene_w, jnp.int32(0))

        for c in range(_NCHUNKS):
            for cp in rcps:
                cp.wait()
            if c + 1 < _NCHUNKS:
                rcps = issue_rows(c + 1)
            rows_v = rows[c % 2]

            def gene_body(gi, carry):
                gb = gi * K
                sb = c * _ROWS + gb
                ws = [wts_v[pl.ds(sb + 16 * q, 16)] for q in range(4)]
                accs = [jnp.zeros((16,), jnp.float32) for _ in range(8)]
                for q in range(4):
                    for u in range(16):
                        ek = ws[q][u]
                        for j in range(8):
                            accs[j] = accs[j] + ek * rows_v[
                                gb + q * 16 + u, pl.ds(16 * j, 16)]
                orow = c * _CHUNK_GENES + gi
                for j in range(8):
                    out_v[orow, pl.ds(16 * j, 16)] = accs[j]
                return carry

            lax.fori_loop(0, _CHUNK_GENES, gene_body, jnp.int32(0))

    pl.run_scoped(phase2,
                  pltpu.VMEM((_ROWS, D), jnp.float32),
                  pltpu.VMEM((_ROWS, D), jnp.float32))

    pltpu.sync_copy(out_v, out_hbm.at[pl.ds(wid * _GPT, _GPT)])


def _sc_pool(mask2d, logits, spot_emb):
    mesh = plsc.VectorSubcoreMesh(core_axis_name="c", subcore_axis_name="s")
    f = pl.kernel(
        _sc_pool_body,
        out_type=jax.ShapeDtypeStruct((N_GENES, D), jnp.float32),
        mesh=mesh,
        scratch_types=[
            pltpu.VMEM((_GPT, K), jnp.int32),
            pltpu.VMEM((_SLOTS,), jnp.int32),
            pltpu.VMEM((_SLOTS,), jnp.float32),
            pltpu.VMEM((_SLOTS,), jnp.float32),
            pltpu.VMEM((_SLOTS,), jnp.float32),
            pltpu.VMEM((_GPT, D), jnp.float32),
            pltpu.SemaphoreType.DMA,
            pltpu.SemaphoreType.DMA,
            pltpu.SemaphoreType.DMA,
        ],
        compiler_params=pltpu.CompilerParams(needs_layout_passes=False),
    )
    return f(mask2d, logits, spot_emb)


def kernel(spot_emb, gene_spot_mask, W_w, W_b, v_w):
    logits = _spot_logits(spot_emb, W_w, W_b.reshape(1, D),
                          v_w.reshape(1, D))
    return _sc_pool(gene_spot_mask.astype(jnp.int32), logits, spot_emb)


# speedup vs baseline: 1.5201x; 1.2823x over previous
import functools

import jax
import jax.numpy as jnp
from jax import lax
from jax.experimental import pallas as pl
from jax.experimental.pallas import tpu as pltpu
from jax.experimental.pallas import tpu_sc as plsc

N_SPOTS = 50000
D = 128
N_GENES = 512
K = 64

_LOGITS_BLOCK = 10240
_N_PAD = 5 * _LOGITS_BLOCK


def _logits_body(x_ref, w_ref, b_ref, v_ref, o_ref):
    x = x_ref[...]
    h = jnp.tanh(
        lax.dot_general(
            x, w_ref[...], (((1,), (1,)), ((), ())),
            preferred_element_type=jnp.float32,
        )
        + b_ref[...]
    )
    hv = (h * v_ref[...]).reshape(_LOGITS_BLOCK // 128, 128, D)
    o_ref[...] = jnp.sum(hv, axis=2)


def _spot_logits(spot_emb, W_w, W_b2, v_w):
    return pl.pallas_call(
        _logits_body,
        grid=(5,),
        in_specs=[
            pl.BlockSpec((_LOGITS_BLOCK, D), lambda i: (i, 0)),
            pl.BlockSpec((D, D), lambda i: (0, 0)),
            pl.BlockSpec((1, D), lambda i: (0, 0)),
            pl.BlockSpec((1, D), lambda i: (0, 0)),
        ],
        out_specs=pl.BlockSpec((_LOGITS_BLOCK // 128, 128), lambda i: (i, 0)),
        out_shape=jax.ShapeDtypeStruct((_N_PAD // 128, 128), jnp.float32),
    )(spot_emb, W_w, W_b2, v_w)


_NTILES = 32
_GPT = N_GENES // _NTILES
_SLOTS = _GPT * K
_CHUNK_GENES = 4
_ROWS = _CHUNK_GENES * K
_NCHUNKS = _GPT // _CHUNK_GENES
_IDXCH = 128


def _sc_pool_body(mask_hbm, logits_hbm, emb_hbm, out_hbm,
                  ids2_v, ids_v, lg_v, val_v, wts_v, out_v,
                  sem_l, sem_r0, sem_r1):
    nc = 2
    wid = lax.axis_index("s") * nc + lax.axis_index("c")

    pltpu.sync_copy(mask_hbm.at[pl.ds(wid * _GPT, _GPT)], ids2_v)
    for r in range(_GPT):
        for q in range(4):
            ids_v[pl.ds(r * K + 16 * q, 16)] = ids2_v[r, pl.ds(16 * q, 16)]

    lcps = [
        pltpu.async_copy(
            logits_hbm.at[ids_v.at[pl.ds(i * _IDXCH, _IDXCH)]],
            lg_v.at[pl.ds(i * _IDXCH, _IDXCH)], sem_l)
        for i in range(_SLOTS // _IDXCH)
    ]

    def dedup(table_v):
        def gene_d(g, carry):
            sb = g * K
            ids = [ids_v[pl.ds(sb + 16 * q, 16)] for q in range(4)]
            pay = [lax.iota(jnp.int32, 16) + 16 * q for q in range(4)]
            for q in range(4):
                plsc.store_scatter(table_v, [ids[q]], pay[q])
            for q in range(4):
                ok = plsc.load_gather(table_v, [ids[q]]) == pay[q]
                val_v[pl.ds(sb + 16 * q, 16)] = jnp.where(
                    ok, jnp.float32(1.0), jnp.float32(0.0))
            return carry

        lax.fori_loop(0, _GPT, gene_d, jnp.int32(0))

    pl.run_scoped(dedup, pltpu.VMEM((N_SPOTS,), jnp.int32))

    def phase2(rows0_v, rows1_v):
        rows = [rows0_v, rows1_v]
        sems = [sem_r0, sem_r1]

        def issue_rows(c):
            buf = rows[c % 2]
            sem = sems[c % 2]
            return [
                pltpu.async_copy(
                    emb_hbm.at[ids_v.at[pl.ds(c * _ROWS + i * _IDXCH,
                                              _IDXCH)]],
                    buf.at[pl.ds(i * _IDXCH, _IDXCH)], sem)
                for i in range(_ROWS // _IDXCH)
            ]

        rcps = issue_rows(0)
        for cp in lcps:
            cp.wait()

        def gene_w(g, carry):
            sb = g * K
            ls = [lg_v[pl.ds(sb + 16 * q, 16)] for q in range(4)]
            va = [val_v[pl.ds(sb + 16 * q, 16)] > jnp.float32(0.5)
                  for q in range(4)]
            lm = [jnp.where(va[q], ls[q], jnp.float32(-1e30))
                  for q in range(4)]
            m = jnp.max(jnp.maximum(jnp.maximum(lm[0], lm[1]),
                                    jnp.maximum(lm[2], lm[3])))
            es = [jnp.where(va[q], jnp.exp(ls[q] - m), jnp.float32(0.0))
                  for q in range(4)]
            s = jnp.sum(es[0] + es[1] + es[2] + es[3])
            inv_v = jnp.full((16,), 1.0, jnp.float32) / jnp.full(
                (16,), s, jnp.float32)
            for q in range(4):
                wts_v[pl.ds(sb + 16 * q, 16)] = es[q] * inv_v
            return carry

        lax.fori_loop(0, _GPT, gene_w, jnp.int32(0))

        for c in range(_NCHUNKS):
            for cp in rcps:
                cp.wait()
            if c + 1 < _NCHUNKS:
                rcps = issue_rows(c + 1)
            rows_v = rows[c % 2]

            def gene_body(gi, carry):
                gb = gi * K
                sb = c * _ROWS + gb
                ws = [wts_v[pl.ds(sb + 16 * q, 16)] for q in range(4)]
                accs = [jnp.zeros((16,), jnp.float32) for _ in range(8)]
                for q in range(4):
                    for u in range(16):
                        ek = ws[q][u]
                        for j in range(8):
                            accs[j] = accs[j] + ek * rows_v[
                                gb + q * 16 + u, pl.ds(16 * j, 16)]
                orow = c * _CHUNK_GENES + gi
                for j in range(8):
                    out_v[orow, pl.ds(16 * j, 16)] = accs[j]
                return carry

            lax.fori_loop(0, _CHUNK_GENES, gene_body, jnp.int32(0))

    pl.run_scoped(phase2,
                  pltpu.VMEM((_ROWS, D), jnp.float32),
                  pltpu.VMEM((_ROWS, D), jnp.float32))

    pltpu.sync_copy(out_v, out_hbm.at[pl.ds(wid * _GPT, _GPT)])


def _sc_pool(mask2d, logits, spot_emb):
    mesh = plsc.VectorSubcoreMesh(core_axis_name="c", subcore_axis_name="s")
    f = pl.kernel(
        _sc_pool_body,
        out_type=jax.ShapeDtypeStruct((N_GENES, D), jnp.float32),
        mesh=mesh,
        scratch_types=[
            pltpu.VMEM((_GPT, K), jnp.int32),
            pltpu.VMEM((_SLOTS,), jnp.int32),
            pltpu.VMEM((_SLOTS,), jnp.float32),
            pltpu.VMEM((_SLOTS,), jnp.float32),
            pltpu.VMEM((_SLOTS,), jnp.float32),
            pltpu.VMEM((_GPT, D), jnp.float32),
            pltpu.SemaphoreType.DMA,
            pltpu.SemaphoreType.DMA,
            pltpu.SemaphoreType.DMA,
        ],
        compiler_params=pltpu.CompilerParams(needs_layout_passes=False),
    )
    return f(mask2d, logits, spot_emb)


def kernel(spot_emb, gene_spot_mask, W_w, W_b, v_w):
    logits = _spot_logits(spot_emb, W_w, W_b.reshape(1, D),
                          v_w.reshape(1, D)).reshape(_N_PAD)
    return _sc_pool(gene_spot_mask.astype(jnp.int32), logits, spot_emb)
